# baseline (speedup 1.0000x reference)
import functools

import jax
import jax.numpy as jnp
import numpy as np
from jax import lax
from jax.experimental import pallas as pl
from jax.experimental.pallas import tpu as pltpu
from jax.experimental.pallas import tpu_sc as plsc

NUM_FIELDS = 26
VOCAB = 100000
EMBED_DIM = 16
BATCH = 1024
FLAT_DIM = NUM_FIELDS * EMBED_DIM
PAIR_NUM = (NUM_FIELDS * (NUM_FIELDS - 1)) // 2
PAIR_PAD = 328
EXP_DIM = PAIR_PAD * EMBED_DIM
NCHUNK = EXP_DIM // 128

ROWS = BATCH * NUM_FIELDS
NW = 32
ROWS_PER_W = ROWS // NW
GCHUNK = 104
NG = ROWS_PER_W // GCHUNK


def _sc_gather_body(table_hbm, idx_hbm, out_hbm, idx_v, rows_v, sem):
    wid = lax.axis_index("s") * 2 + lax.axis_index("c")
    base = wid * ROWS_PER_W
    pltpu.sync_copy(idx_hbm.at[wid], idx_v)
    copies = []
    for g in range(NG):
        copies.append(
            pltpu.async_copy(
                table_hbm.at[idx_v.at[g]],
                rows_v.at[pl.ds(g * GCHUNK, GCHUNK)],
                sem,
            )
        )
    for cp in copies:
        cp.wait()
    pltpu.sync_copy(rows_v, out_hbm.at[pl.ds(base, ROWS_PER_W)])


def _sc_gather(table_flat, flat_idx):
    mesh = plsc.VectorSubcoreMesh(core_axis_name="c", subcore_axis_name="s")
    kern = functools.partial(
        pl.kernel,
        mesh=mesh,
        compiler_params=pltpu.CompilerParams(use_tc_tiling_on_sc=False),
        out_type=jax.ShapeDtypeStruct((ROWS, EMBED_DIM), jnp.float32),
        scratch_types=[
            pltpu.VMEM((NG, GCHUNK), jnp.int32),
            pltpu.VMEM((ROWS_PER_W, EMBED_DIM), jnp.float32),
            pltpu.SemaphoreType.DMA,
        ],
    )(_sc_gather_body)
    return kern(table_flat, flat_idx.reshape(NW, NG, GCHUNK))


def _tc_body(flat_ref, wbd_ref, w1a_ref, w1b_ref, w1c_ref, b1_ref,
             w2_ref, b2_ref, w3_ref, b3_ref, wout_ref, bout_ref, out_ref):
    bb = flat_ref.shape[0]
    flat = flat_ref[:]
    pad = jnp.zeros((bb, EXP_DIM - PAIR_NUM * EMBED_DIM), jnp.float32)
    q_parts = [flat[:, EMBED_DIM * (f + 1):] for f in range(NUM_FIELDS - 1)]
    q_parts.append(pad)
    q = jnp.concatenate(q_parts, axis=1)
    p_parts = []
    for f in range(NUM_FIELDS - 1):
        seg = flat[:, EMBED_DIM * f:EMBED_DIM * (f + 1)]
        p_parts.extend([seg] * (NUM_FIELDS - 1 - f))
    p_parts.append(pad)
    p = jnp.concatenate(p_parts, axis=1)

    t_parts = []
    for c in range(NCHUNK):
        t_parts.append(
            lax.dot_general(
                p[:, 128 * c:128 * (c + 1)], wbd_ref[c],
                (((1,), (0,)), ((), ())),
                preferred_element_type=jnp.float32,
            )
        )
    t = jnp.concatenate(t_parts, axis=1)

    u = p * q
    v = t * q

    mm = functools.partial(lax.dot_general, dimension_numbers=(((1,), (0,)), ((), ())),
                           preferred_element_type=jnp.float32)
    h = (mm(flat, w1a_ref[:]) + mm(u, w1b_ref[:]) + mm(v, w1c_ref[:])
         + b1_ref[:])
    h = jnp.maximum(h, 0.0)
    h = jnp.maximum(mm(h, w2_ref[:]) + b2_ref[:], 0.0)
    h = jnp.maximum(mm(h, w3_ref[:]) + b3_ref[:], 0.0)
    out_ref[:] = jax.nn.sigmoid(mm(h, wout_ref[:]) + bout_ref[:])


def _tc_forward(flat, wbd, w1a, w1b_exp, w1c_exp, b1, W2, b2, W3, b3, Wout, bout,
                block_b=256):
    nsteps = BATCH // block_b
    full = lambda shape: pl.BlockSpec(shape, lambda i: tuple(0 for _ in shape))
    return pl.pallas_call(
        _tc_body,
        grid=(nsteps,),
        in_specs=[
            pl.BlockSpec((block_b, FLAT_DIM), lambda i: (i, 0)),
            full((NCHUNK, 128, 128)),
            full((FLAT_DIM, 400)),
            full((EXP_DIM, 400)),
            full((EXP_DIM, 400)),
            full((1, 400)),
            full((400, 400)),
            full((1, 400)),
            full((400, 400)),
            full((1, 400)),
            full((400, 1)),
            full((1, 1)),
        ],
        out_specs=pl.BlockSpec((block_b, 1), lambda i: (i, 0)),
        out_shape=jax.ShapeDtypeStruct((BATCH, 1), jnp.float32),
    )(flat, wbd, w1a, w1b_exp, w1c_exp, b1.reshape(1, 400), W2,
      b2.reshape(1, 400), W3, b3.reshape(1, 400), Wout, bout.reshape(1, 1))


def _prep_weights(w_outer, W1):
    wt = jnp.transpose(w_outer, (1, 2, 0))
    wt = jnp.pad(wt, ((0, PAIR_PAD - PAIR_NUM), (0, 0), (0, 0)))
    blocks = wt.reshape(NCHUNK, 8, EMBED_DIM, EMBED_DIM)
    eye8 = jnp.eye(8, dtype=jnp.float32)
    wbd = (blocks[:, :, :, None, :] * eye8[None, :, None, :, None]).reshape(
        NCHUNK, 128, 128)
    w1a = W1[:FLAT_DIM]
    w1b = W1[FLAT_DIM:FLAT_DIM + PAIR_NUM]
    w1c = W1[FLAT_DIM + PAIR_NUM:]
    w1b_exp = jnp.pad(jnp.repeat(w1b, EMBED_DIM, axis=0),
                      ((0, EXP_DIM - PAIR_NUM * EMBED_DIM), (0, 0)))
    w1c_exp = jnp.pad(jnp.repeat(w1c, EMBED_DIM, axis=0),
                      ((0, EXP_DIM - PAIR_NUM * EMBED_DIM), (0, 0)))
    return wbd, w1a, w1b_exp, w1c_exp


def kernel(inputs, tables, w_outer, W1, b1, W2, b2, W3, b3, Wout, bout):
    sparse = inputs[:, 13:].astype(jnp.int32)
    flat_idx = (sparse + VOCAB * jnp.arange(NUM_FIELDS, dtype=jnp.int32)[None, :]
                ).reshape(ROWS)
    table_flat = tables.reshape(NUM_FIELDS * VOCAB, EMBED_DIM)
    rows = _sc_gather(table_flat, flat_idx)
    flat = rows.reshape(BATCH, FLAT_DIM)
    wbd, w1a, w1b_exp, w1c_exp = _prep_weights(w_outer, W1)
    return _tc_forward(flat, wbd, w1a, w1b_exp, w1c_exp,
                       b1, W2, b2, W3, b3, Wout, bout)
